# baseline (speedup 1.0000x reference)
import functools

import jax
import jax.numpy as jnp
from jax import lax
from jax.experimental import pallas as pl
from jax.experimental.pallas import tpu as pltpu
from jax.experimental.pallas import tpu_sc as plsc

N_NODES = 10000
N_EDGES = 320000
CH = 128
NEG_SLOPE = 0.2

NUM_CORES = 2
NUM_SUBCORES = 16
NW = NUM_CORES * NUM_SUBCORES
E_PER = N_EDGES // NW
CHUNK = 80
NCHUNK = E_PER // CHUNK
N_PAD = 10112
N_PER = N_PAD // NUM_SUBCORES

_f32 = jnp.float32
_i32 = jnp.int32


def _k1_body(x_ref, w_ref, a2_ref, xw_ref, s_ref, smax_ref):
    xw = jnp.dot(x_ref[...], w_ref[...], preferred_element_type=_f32)
    xw_ref[...] = xw
    s = jnp.dot(xw, a2_ref[...], preferred_element_type=_f32)
    s_ref[...] = s
    smax_ref[...] = jnp.max(s, axis=0, keepdims=True)


_k1 = pl.pallas_call(
    _k1_body,
    out_shape=(
        jax.ShapeDtypeStruct((N_NODES, CH), _f32),
        jax.ShapeDtypeStruct((N_NODES, 8), _f32),
        jax.ShapeDtypeStruct((1, 8), _f32),
    ),
)

def _k2_body(eat_ref, wet_ref, ae_ref, eat_out_ref, se_ref, semax_ref):
    eat = jnp.dot(wet_ref[...], eat_ref[...], preferred_element_type=_f32)
    eat_out_ref[...] = eat[:4]
    se = jnp.sum(eat * ae_ref[...], axis=0)
    se_ref[...] = se
    semax_ref[...] = jnp.full((1, 8), jnp.max(se), dtype=_f32)


_k2 = pl.pallas_call(
    _k2_body,
    out_shape=(
        jax.ShapeDtypeStruct((4, N_EDGES), _f32),
        jax.ShapeDtypeStruct((N_EDGES,), _f32),
        jax.ShapeDtypeStruct((1, 8), _f32),
    ),
)


def _k7_body(p_ref, dpt_ref, b_ref, o_ref):
    den = dpt_ref[:, 0:1] + dpt_ref[:, 1:2] + 1e-16
    o_ref[...] = (p_ref[0, :N_NODES] + p_ref[1, :N_NODES]) / den + b_ref[...]


_k7 = pl.pallas_call(
    _k7_body,
    out_shape=jax.ShapeDtypeStruct((N_NODES, CH), _f32),
)


_SC_MESH = plsc.VectorSubcoreMesh(core_axis_name="c", subcore_axis_name="s")


def _attn_body(si_hbm, sj_hbm, se_hbm, src_hbm, dst_hbm, m_hbm, zn_hbm,
               w_hbm, dpart_hbm,
               si_v, sj_v, se_v, src_v, dst_v, w_v, m_v, den_sh):
    c = lax.axis_index("c")
    s = lax.axis_index("s")
    wid = c * NUM_SUBCORES + s

    @pl.when(s == 0)
    def _():
        pltpu.sync_copy(zn_hbm, den_sh)

    pltpu.sync_copy(si_hbm, si_v)
    pltpu.sync_copy(sj_hbm, sj_v)
    pltpu.sync_copy(se_hbm.at[pl.ds(wid * E_PER, E_PER)], se_v)
    pltpu.sync_copy(src_hbm.at[wid], src_v)
    pltpu.sync_copy(dst_hbm.at[wid], dst_v)
    pltpu.sync_copy(m_hbm, m_v)
    gmax = m_v[...]
    plsc.subcore_barrier()

    def jbody(j, carry):
        for g in range(CHUNK // 16):
            sl = pl.ds(g * 16, 16)
            di = dst_v[j, sl]
            sri = src_v[j, sl]
            l = (plsc.load_gather(si_v, [di])
                 + plsc.load_gather(sj_v, [sri])
                 + se_v[pl.ds(j * CHUNK + g * 16, 16)])
            l = jnp.where(l >= 0.0, l, l * NEG_SLOPE)
            w_v[j, sl] = jnp.exp(l - gmax)
        pltpu.sync_copy(w_v.at[j], den_sh.at[dst_v.at[j]], add=True)
        return carry

    lax.fori_loop(0, NCHUNK, jbody, 0)
    pltpu.sync_copy(w_v, w_hbm.at[wid])
    plsc.subcore_barrier()

    @pl.when(s == 0)
    def _():
        pltpu.sync_copy(den_sh, dpart_hbm.at[c])


_attn = functools.partial(
    pl.kernel,
    out_type=(
        jax.ShapeDtypeStruct((NW, NCHUNK, CHUNK), _f32),
        jax.ShapeDtypeStruct((NUM_CORES, N_NODES), _f32),
    ),
    mesh=_SC_MESH,
    compiler_params=pltpu.CompilerParams(needs_layout_passes=False),
    scratch_types=[
        pltpu.VMEM((N_NODES,), _f32),
        pltpu.VMEM((N_NODES,), _f32),
        pltpu.VMEM((E_PER,), _f32),
        pltpu.VMEM((NCHUNK, CHUNK), _i32),
        pltpu.VMEM((NCHUNK, CHUNK), _i32),
        pltpu.VMEM((NCHUNK, CHUNK), _f32),
        pltpu.VMEM((16,), _f32),
        pltpu.VMEM_SHARED((N_NODES,), _f32),
    ],
)(_attn_body)


def _aggr_body(xw_hbm, src_hbm, dst_hbm, w_hbm,
               p_hbm,
               srcb, dstb, wb, rows3, aggr_sh, gsem, ssem, psem):
    c = lax.axis_index("c")
    s = lax.axis_index("s")
    wid = c * NUM_SUBCORES + s

    def zbody(i, carry):
        for f in range(CH // 16):
            rows3[0, i, pl.ds(f * 16, 16)] = jnp.zeros((16,), _f32)
        return carry

    lax.fori_loop(0, CHUNK, zbody, 0)
    nfull = N_PER // CHUNK
    for k in range(nfull):
        pltpu.sync_copy(rows3.at[0],
                        aggr_sh.at[pl.ds(s * N_PER + k * CHUNK, CHUNK)])
    rem = N_PER - nfull * CHUNK
    if rem:
        pltpu.sync_copy(rows3.at[0, pl.ds(0, rem)],
                        aggr_sh.at[pl.ds(s * N_PER + nfull * CHUNK, rem)])
    plsc.subcore_barrier()

    pltpu.sync_copy(src_hbm.at[wid, 0], srcb.at[0])
    pltpu.sync_copy(dst_hbm.at[wid, 0], dstb.at[0])
    pltpu.sync_copy(w_hbm.at[wid, 0], wb.at[0])
    pltpu.sync_copy(src_hbm.at[wid, 1], srcb.at[1])
    pltpu.sync_copy(dst_hbm.at[wid, 1], dstb.at[1])
    pltpu.sync_copy(w_hbm.at[wid, 1], wb.at[1])
    pltpu.async_copy(src_hbm.at[wid, 2], srcb.at[2], psem)
    pltpu.async_copy(dst_hbm.at[wid, 2], dstb.at[2], psem)
    pltpu.async_copy(w_hbm.at[wid, 2], wb.at[2], psem)
    pltpu.async_copy(xw_hbm.at[srcb.at[0]], rows3.at[0], gsem.at[0])

    def jbody(j, carry):
        r = lax.rem(j, 3)
        rn = lax.rem(j + 1, 3)
        q = lax.rem(j, 5)
        qn = lax.rem(j + 1, 5)
        qp = lax.rem(j + 3, 5)
        b = lax.rem(j, 2)
        bn = lax.rem(j + 1, 2)
        jpre = jnp.minimum(j + 3, NCHUNK - 1)
        pltpu.make_async_copy(src_hbm.at[wid, 0], srcb.at[q], psem).wait()
        pltpu.make_async_copy(dst_hbm.at[wid, 0], dstb.at[q], psem).wait()
        pltpu.make_async_copy(w_hbm.at[wid, 0], wb.at[q], psem).wait()
        pltpu.make_async_copy(xw_hbm.at[srcb.at[q]], rows3.at[r],
                              gsem.at[b]).wait()
        @pl.when(j >= 2)
        def _():
            pltpu.make_async_copy(rows3.at[rn], aggr_sh.at[dstb.at[qp]],
                                  ssem.at[b]).wait()

        pltpu.async_copy(src_hbm.at[wid, jpre], srcb.at[qp], psem)
        pltpu.async_copy(dst_hbm.at[wid, jpre], dstb.at[qp], psem)
        pltpu.async_copy(w_hbm.at[wid, jpre], wb.at[qp], psem)
        pltpu.async_copy(xw_hbm.at[srcb.at[qn]], rows3.at[rn], gsem.at[bn])

        def ibody(i, icarry):
            a = plsc.load_gather(wb, [jnp.full((16,), q, _i32),
                                      jnp.full((16,), i, _i32)])
            for f in range(CH // 16):
                fl = pl.ds(f * 16, 16)
                rows3[r, i, fl] = rows3[r, i, fl] * a
            return icarry

        lax.fori_loop(0, CHUNK, ibody, 0)
        pltpu.async_copy(rows3.at[r], aggr_sh.at[dstb.at[q]], ssem.at[b],
                         add=True)
        return carry

    lax.fori_loop(0, NCHUNK, jbody, 0)
    pltpu.make_async_copy(src_hbm.at[wid, 0], srcb.at[0], psem).wait()
    pltpu.make_async_copy(dst_hbm.at[wid, 0], dstb.at[0], psem).wait()
    pltpu.make_async_copy(w_hbm.at[wid, 0], wb.at[0], psem).wait()
    pltpu.make_async_copy(xw_hbm.at[srcb.at[0]], rows3.at[NCHUNK % 3],
                          gsem.at[NCHUNK % 2]).wait()
    pltpu.make_async_copy(rows3.at[0], aggr_sh.at[dstb.at[0]],
                          ssem.at[(NCHUNK - 2) % 2]).wait()
    pltpu.make_async_copy(rows3.at[0], aggr_sh.at[dstb.at[0]],
                          ssem.at[(NCHUNK - 1) % 2]).wait()
    plsc.subcore_barrier()
    pltpu.sync_copy(aggr_sh.at[pl.ds(s * N_PER, N_PER)],
                    p_hbm.at[c, pl.ds(s * N_PER, N_PER)])


_aggr = functools.partial(
    pl.kernel,
    out_type=jax.ShapeDtypeStruct((NUM_CORES, N_PAD, CH), _f32),
    mesh=_SC_MESH,
    compiler_params=pltpu.CompilerParams(needs_layout_passes=False),
    scratch_types=[
        pltpu.VMEM((5, CHUNK), _i32),
        pltpu.VMEM((5, CHUNK), _i32),
        pltpu.VMEM((5, CHUNK), _f32),
        pltpu.VMEM((3, CHUNK, CH), _f32),
        pltpu.VMEM_SHARED((N_PAD, CH), _f32),
        pltpu.SemaphoreType.DMA((2,)),
        pltpu.SemaphoreType.DMA((2,)),
        pltpu.SemaphoreType.DMA,
    ],
)(_aggr_body)


@jax.jit
def kernel(x, edge_index, edge_attr, W, We, att, bias):
    src = edge_index[0].astype(_i32).reshape(NW, NCHUNK, CHUNK)
    dst = edge_index[1].astype(_i32).reshape(NW, NCHUNK, CHUNK)
    attf = att.reshape(2 * CH + 4)
    a2 = jnp.pad(jnp.stack([attf[:CH], attf[CH:2 * CH]], axis=1),
                 ((0, 0), (0, 6)))
    wet = jnp.pad(We.T, ((0, 4), (0, 0)))
    ae = jnp.pad(attf[2 * CH:].reshape(4, 1), ((0, 4), (0, 0)))

    xw, s, smax = _k1(x, W, a2)
    eat, se, semax = _k2(edge_attr.T, wet, ae)
    ea = eat.T
    s_i = s[:, 0]
    s_j = s[:, 1]

    t = smax[0, 0] + smax[0, 1] + semax[0, 0]
    m = jnp.where(t >= 0.0, t, NEG_SLOPE * t)
    m_arr = jnp.full((16,), m, dtype=_f32)
    zn = jnp.zeros((N_NODES,), dtype=_f32)

    w2d, dpart = _attn(s_i, s_j, se, src, dst, m_arr, zn)
    parts = _aggr(xw, src, dst, w2d)
    out = _k7(parts, dpart.T, bias.reshape(1, CH))
    return out, edge_index, ea


# speedup vs baseline: 1.2819x; 1.2819x over previous
import functools

import jax
import jax.numpy as jnp
from jax import lax
from jax.experimental import pallas as pl
from jax.experimental.pallas import tpu as pltpu
from jax.experimental.pallas import tpu_sc as plsc

N_NODES = 10000
N_EDGES = 320000
CH = 128
NEG_SLOPE = 0.2

NUM_CORES = 2
NUM_SUBCORES = 16
NW = NUM_CORES * NUM_SUBCORES
E_PER = N_EDGES // NW
CHUNK = 80
NCHUNK = E_PER // CHUNK
N_PAD = 10112
N_PER = N_PAD // NUM_SUBCORES

_f32 = jnp.float32
_i32 = jnp.int32


def _k1_body(x_ref, w_ref, a2_ref, xw_ref, s_ref, smax_ref):
    xw = jnp.dot(x_ref[...], w_ref[...], preferred_element_type=_f32)
    xw_ref[...] = xw
    s = jnp.dot(xw, a2_ref[...], preferred_element_type=_f32)
    s_ref[...] = s
    smax_ref[...] = jnp.max(s, axis=0, keepdims=True)


_k1 = pl.pallas_call(
    _k1_body,
    out_shape=(
        jax.ShapeDtypeStruct((N_NODES, CH), _f32),
        jax.ShapeDtypeStruct((N_NODES, 8), _f32),
        jax.ShapeDtypeStruct((1, 8), _f32),
    ),
)

def _k2_body(eat_ref, wet_ref, ae_ref, eat_out_ref, se_ref, semax_ref):
    eat = jnp.dot(wet_ref[...], eat_ref[...], preferred_element_type=_f32)
    eat_out_ref[...] = eat[:4]
    se = jnp.sum(eat * ae_ref[...], axis=0)
    se_ref[...] = se
    semax_ref[...] = jnp.full((1, 8), jnp.max(se), dtype=_f32)


_k2 = pl.pallas_call(
    _k2_body,
    out_shape=(
        jax.ShapeDtypeStruct((4, N_EDGES), _f32),
        jax.ShapeDtypeStruct((N_EDGES,), _f32),
        jax.ShapeDtypeStruct((1, 8), _f32),
    ),
)


def _k7_body(p_ref, dpt_ref, b_ref, o_ref):
    den = dpt_ref[:, 0:1] + dpt_ref[:, 1:2] + 1e-16
    o_ref[...] = (p_ref[0, :N_NODES] + p_ref[1, :N_NODES]) / den + b_ref[...]


_k7 = pl.pallas_call(
    _k7_body,
    out_shape=jax.ShapeDtypeStruct((N_NODES, CH), _f32),
)


_SC_MESH = plsc.VectorSubcoreMesh(core_axis_name="c", subcore_axis_name="s")


def _attn_body(si_hbm, sj_hbm, se_hbm, src_hbm, dst_hbm, m_hbm, zn_hbm,
               w_hbm, dpart_hbm,
               si_v, sj_v, se_v, src_v, dst_v, w_v, m_v, den_sh):
    c = lax.axis_index("c")
    s = lax.axis_index("s")
    wid = c * NUM_SUBCORES + s

    @pl.when(s == 0)
    def _():
        pltpu.sync_copy(zn_hbm, den_sh)

    pltpu.sync_copy(si_hbm, si_v)
    pltpu.sync_copy(sj_hbm, sj_v)
    pltpu.sync_copy(se_hbm.at[pl.ds(wid * E_PER, E_PER)], se_v)
    pltpu.sync_copy(src_hbm.at[wid], src_v)
    pltpu.sync_copy(dst_hbm.at[wid], dst_v)
    pltpu.sync_copy(m_hbm, m_v)
    gmax = m_v[...]
    plsc.subcore_barrier()

    def jbody(j, carry):
        for g in range(CHUNK // 16):
            sl = pl.ds(g * 16, 16)
            di = dst_v[j, sl]
            sri = src_v[j, sl]
            l = (plsc.load_gather(si_v, [di])
                 + plsc.load_gather(sj_v, [sri])
                 + se_v[pl.ds(j * CHUNK + g * 16, 16)])
            l = jnp.where(l >= 0.0, l, l * NEG_SLOPE)
            w_v[j, sl] = jnp.exp(l - gmax)
        pltpu.sync_copy(w_v.at[j], den_sh.at[dst_v.at[j]], add=True)
        return carry

    lax.fori_loop(0, NCHUNK, jbody, 0)
    pltpu.sync_copy(w_v, w_hbm.at[wid])
    plsc.subcore_barrier()

    @pl.when(s == 0)
    def _():
        pltpu.sync_copy(den_sh, dpart_hbm.at[c])


_attn = functools.partial(
    pl.kernel,
    out_type=(
        jax.ShapeDtypeStruct((NW, NCHUNK, CHUNK), _f32),
        jax.ShapeDtypeStruct((NUM_CORES, N_NODES), _f32),
    ),
    mesh=_SC_MESH,
    compiler_params=pltpu.CompilerParams(needs_layout_passes=False),
    scratch_types=[
        pltpu.VMEM((N_NODES,), _f32),
        pltpu.VMEM((N_NODES,), _f32),
        pltpu.VMEM((E_PER,), _f32),
        pltpu.VMEM((NCHUNK, CHUNK), _i32),
        pltpu.VMEM((NCHUNK, CHUNK), _i32),
        pltpu.VMEM((NCHUNK, CHUNK), _f32),
        pltpu.VMEM((16,), _f32),
        pltpu.VMEM_SHARED((N_NODES,), _f32),
    ],
)(_attn_body)


def _aggr_body(xw_hbm, src_hbm, dst_hbm, w_hbm, zr_hbm,
               p_hbm,
               src_v, dst_v, w_c, rows_v, aggr_sh, sem):
    c = lax.axis_index("c")
    s = lax.axis_index("s")
    wid = c * NUM_SUBCORES + s

    pltpu.sync_copy(zr_hbm, aggr_sh.at[pl.ds(s * N_PER, N_PER)])
    pltpu.sync_copy(src_hbm.at[wid], src_v)
    pltpu.sync_copy(dst_hbm.at[wid], dst_v)
    plsc.subcore_barrier()

    def jbody(j, carry):
        pltpu.sync_copy(w_hbm.at[wid, j], w_c)
        pltpu.async_copy(xw_hbm.at[src_v.at[j]], rows_v, sem).wait()

        def gbody(g, gcarry):
            w16 = w_c[pl.ds(g * 16, 16)]
            for k in range(16):
                a = w16[k]
                row = g * 16 + k
                for f in range(CH // 16):
                    fl = pl.ds(f * 16, 16)
                    rows_v[row, fl] = rows_v[row, fl] * a
            return gcarry

        lax.fori_loop(0, CHUNK // 16, gbody, 0)
        pltpu.sync_copy(rows_v, aggr_sh.at[dst_v.at[j]], add=True)
        return carry

    lax.fori_loop(0, NCHUNK, jbody, 0)
    plsc.subcore_barrier()
    pltpu.sync_copy(aggr_sh.at[pl.ds(s * N_PER, N_PER)],
                    p_hbm.at[c, pl.ds(s * N_PER, N_PER)])


_aggr = functools.partial(
    pl.kernel,
    out_type=jax.ShapeDtypeStruct((NUM_CORES, N_PAD, CH), _f32),
    mesh=_SC_MESH,
    compiler_params=pltpu.CompilerParams(needs_layout_passes=False),
    scratch_types=[
        pltpu.VMEM((NCHUNK, CHUNK), _i32),
        pltpu.VMEM((NCHUNK, CHUNK), _i32),
        pltpu.VMEM((CHUNK,), _f32),
        pltpu.VMEM((CHUNK, CH), _f32),
        pltpu.VMEM_SHARED((N_PAD, CH), _f32),
        pltpu.SemaphoreType.DMA,
    ],
)(_aggr_body)


@jax.jit
def kernel(x, edge_index, edge_attr, W, We, att, bias):
    src = edge_index[0].astype(_i32).reshape(NW, NCHUNK, CHUNK)
    dst = edge_index[1].astype(_i32).reshape(NW, NCHUNK, CHUNK)
    attf = att.reshape(2 * CH + 4)
    a2 = jnp.pad(jnp.stack([attf[:CH], attf[CH:2 * CH]], axis=1),
                 ((0, 0), (0, 6)))
    wet = jnp.pad(We.T, ((0, 4), (0, 0)))
    ae = jnp.pad(attf[2 * CH:].reshape(4, 1), ((0, 4), (0, 0)))

    xw, s, smax = _k1(x, W, a2)
    eat, se, semax = _k2(edge_attr.T, wet, ae)
    ea = eat.T
    s_i = s[:, 0]
    s_j = s[:, 1]

    t = smax[0, 0] + smax[0, 1] + semax[0, 0]
    m = jnp.where(t >= 0.0, t, NEG_SLOPE * t)
    m_arr = jnp.full((16,), m, dtype=_f32)
    zn = jnp.zeros((N_NODES,), dtype=_f32)
    zr = jnp.zeros((N_PER, CH), dtype=_f32)

    w2d, dpart = _attn(s_i, s_j, se, src, dst, m_arr, zn)
    parts = _aggr(xw, src, dst, w2d, zr)
    out = _k7(parts, dpart.T, bias.reshape(1, CH))
    return out, edge_index, ea


# speedup vs baseline: 1.8822x; 1.4683x over previous
import functools

import jax
import jax.numpy as jnp
from jax import lax
from jax.experimental import pallas as pl
from jax.experimental.pallas import tpu as pltpu
from jax.experimental.pallas import tpu_sc as plsc

N_NODES = 10000
N_EDGES = 320000
CH = 128
NEG_SLOPE = 0.2

NUM_CORES = 2
NUM_SUBCORES = 16
NW = NUM_CORES * NUM_SUBCORES
E_PER = N_EDGES // NW
CHUNK = 80
NCHUNK = E_PER // CHUNK
N_PAD = 10112
N_PER = N_PAD // NUM_SUBCORES

_f32 = jnp.float32
_i32 = jnp.int32


def _k1_body(x_ref, w_ref, a2_ref, xw_ref, s_ref, smax_ref):
    xw = jnp.dot(x_ref[...], w_ref[...], preferred_element_type=_f32)
    xw_ref[...] = xw
    s = jnp.dot(xw, a2_ref[...], preferred_element_type=_f32)
    s_ref[...] = s
    smax_ref[...] = jnp.max(s, axis=0, keepdims=True)


_k1 = pl.pallas_call(
    _k1_body,
    out_shape=(
        jax.ShapeDtypeStruct((N_NODES, CH), _f32),
        jax.ShapeDtypeStruct((N_NODES, 8), _f32),
        jax.ShapeDtypeStruct((1, 8), _f32),
    ),
)

def _k2_body(eat_ref, wet_ref, ae_ref, eat_out_ref, se_ref, semax_ref):
    eat = jnp.dot(wet_ref[...], eat_ref[...], preferred_element_type=_f32)
    eat_out_ref[...] = eat[:4]
    se = jnp.sum(eat * ae_ref[...], axis=0)
    se_ref[...] = se
    semax_ref[...] = jnp.full((1, 8), jnp.max(se), dtype=_f32)


_k2 = pl.pallas_call(
    _k2_body,
    out_shape=(
        jax.ShapeDtypeStruct((4, N_EDGES), _f32),
        jax.ShapeDtypeStruct((N_EDGES,), _f32),
        jax.ShapeDtypeStruct((1, 8), _f32),
    ),
)


def _k7_body(p_ref, dpt_ref, b_ref, o_ref):
    den = dpt_ref[:, 0:1] + dpt_ref[:, 1:2] + 1e-16
    o_ref[...] = (p_ref[0, :N_NODES] + p_ref[1, :N_NODES]) / den + b_ref[...]


_k7 = pl.pallas_call(
    _k7_body,
    out_shape=jax.ShapeDtypeStruct((N_NODES, CH), _f32),
)


_SC_MESH = plsc.VectorSubcoreMesh(core_axis_name="c", subcore_axis_name="s")


def _attn_body(si_hbm, sj_hbm, se_hbm, src_hbm, dst_hbm, m_hbm, zn_hbm,
               w_hbm, dpart_hbm,
               si_v, sj_v, se_v, src_v, dst_v, w_v, m_v, den_sh):
    c = lax.axis_index("c")
    s = lax.axis_index("s")
    wid = c * NUM_SUBCORES + s

    @pl.when(s == 0)
    def _():
        pltpu.sync_copy(zn_hbm, den_sh)

    pltpu.sync_copy(si_hbm, si_v)
    pltpu.sync_copy(sj_hbm, sj_v)
    pltpu.sync_copy(se_hbm.at[pl.ds(wid * E_PER, E_PER)], se_v)
    pltpu.sync_copy(src_hbm.at[wid], src_v)
    pltpu.sync_copy(dst_hbm.at[wid], dst_v)
    pltpu.sync_copy(m_hbm, m_v)
    gmax = m_v[...]
    plsc.subcore_barrier()

    def jbody(j, carry):
        for g in range(CHUNK // 16):
            sl = pl.ds(g * 16, 16)
            di = dst_v[j, sl]
            sri = src_v[j, sl]
            l = (plsc.load_gather(si_v, [di])
                 + plsc.load_gather(sj_v, [sri])
                 + se_v[pl.ds(j * CHUNK + g * 16, 16)])
            l = jnp.where(l >= 0.0, l, l * NEG_SLOPE)
            w_v[j, sl] = jnp.exp(l - gmax)
        pltpu.sync_copy(w_v.at[j], den_sh.at[dst_v.at[j]], add=True)
        return carry

    lax.fori_loop(0, NCHUNK, jbody, 0)
    pltpu.sync_copy(w_v, w_hbm.at[wid])
    plsc.subcore_barrier()

    @pl.when(s == 0)
    def _():
        pltpu.sync_copy(den_sh, dpart_hbm.at[c])


_attn = functools.partial(
    pl.kernel,
    out_type=(
        jax.ShapeDtypeStruct((NW, NCHUNK, CHUNK), _f32),
        jax.ShapeDtypeStruct((NUM_CORES, N_NODES), _f32),
    ),
    mesh=_SC_MESH,
    compiler_params=pltpu.CompilerParams(needs_layout_passes=False),
    scratch_types=[
        pltpu.VMEM((N_NODES,), _f32),
        pltpu.VMEM((N_NODES,), _f32),
        pltpu.VMEM((E_PER,), _f32),
        pltpu.VMEM((NCHUNK, CHUNK), _i32),
        pltpu.VMEM((NCHUNK, CHUNK), _i32),
        pltpu.VMEM((NCHUNK, CHUNK), _f32),
        pltpu.VMEM((16,), _f32),
        pltpu.VMEM_SHARED((N_NODES,), _f32),
    ],
)(_attn_body)


NSUP = (NCHUNK - 1) // 2


def _aggr_body(xw_hbm, src_hbm, dst_hbm, w_hbm, zr_hbm,
               p_hbm,
               srcb, dstb, wbuf, rows2, aggr_sh, gsem, ssem, psem):
    c = lax.axis_index("c")
    s = lax.axis_index("s")
    wid = c * NUM_SUBCORES + s
    ebase = wid * E_PER

    pltpu.sync_copy(zr_hbm, aggr_sh.at[pl.ds(s * N_PER, N_PER)])
    plsc.subcore_barrier()

    def scale_chunk(k, wslot):
        def gbody(g, gcarry):
            w16 = wbuf[wslot, pl.ds(g * 16, 16)]
            for kk in range(16):
                a = w16[kk]
                row = g * 16 + kk
                for f in range(CH // 16):
                    fl = pl.ds(f * 16, 16)
                    rows2[k, row, fl] = rows2[k, row, fl] * a
            return gcarry

        lax.fori_loop(0, CHUNK // 16, gbody, 0)

    def fetch_idx(row, slot, issue):
        off = ebase + row * CHUNK
        if issue:
            pltpu.async_copy(src_hbm.at[pl.ds(off, CHUNK)], srcb.at[slot],
                             psem)
            pltpu.async_copy(dst_hbm.at[pl.ds(off, CHUNK)], dstb.at[slot],
                             psem)
            pltpu.async_copy(w_hbm.at[pl.ds(off, CHUNK)], wbuf.at[slot],
                             psem)
        else:
            pltpu.make_async_copy(src_hbm.at[pl.ds(0, CHUNK)], srcb.at[slot],
                                  psem).wait()
            pltpu.make_async_copy(dst_hbm.at[pl.ds(0, CHUNK)], dstb.at[slot],
                                  psem).wait()
            pltpu.make_async_copy(w_hbm.at[pl.ds(0, CHUNK)], wbuf.at[slot],
                                  psem).wait()

    fetch_idx(0, 0, True)
    fetch_idx(1, 1, True)

    def jbody(J, carry):
        pb = lax.rem(J, 2)
        pn = 1 - pb
        sA = 2 * pb
        sB = 2 * pb + 1
        nA = 2 * pn
        nB = 2 * pn + 1
        nxt = jnp.minimum(2 * J + 2, NCHUNK - 2)
        fetch_idx(0, sA, False)
        fetch_idx(0, sB, False)

        @pl.when(J > 0)
        def _():
            pltpu.make_async_copy(rows2.at[0], aggr_sh.at[dstb.at[nA]],
                                  ssem).wait()
            pltpu.make_async_copy(rows2.at[1], aggr_sh.at[dstb.at[nB]],
                                  ssem).wait()

        pltpu.async_copy(xw_hbm.at[srcb.at[sA]], rows2.at[0], gsem)
        pltpu.async_copy(xw_hbm.at[srcb.at[sB]], rows2.at[1], gsem)
        fetch_idx(nxt, nA, True)
        fetch_idx(nxt + 1, nB, True)

        pltpu.make_async_copy(xw_hbm.at[srcb.at[sA]], rows2.at[0],
                              gsem).wait()
        scale_chunk(0, sA)
        pltpu.async_copy(rows2.at[0], aggr_sh.at[dstb.at[sA]], ssem,
                         add=True)
        pltpu.make_async_copy(xw_hbm.at[srcb.at[sB]], rows2.at[1],
                              gsem).wait()
        scale_chunk(1, sB)
        pltpu.async_copy(rows2.at[1], aggr_sh.at[dstb.at[sB]], ssem,
                         add=True)
        return carry

    lax.fori_loop(0, NSUP, jbody, 0)
    lastA = 2 * lax.rem(NSUP - 1, 2)
    pltpu.make_async_copy(rows2.at[0], aggr_sh.at[dstb.at[lastA]],
                          ssem).wait()
    pltpu.make_async_copy(rows2.at[1], aggr_sh.at[dstb.at[lastA + 1]],
                          ssem).wait()
    fetch_idx(0, 0, False)
    fetch_idx(0, 1, False)

    toff = ebase + (NCHUNK - 1) * CHUNK
    pltpu.sync_copy(src_hbm.at[pl.ds(toff, CHUNK)], srcb.at[0])
    pltpu.sync_copy(dst_hbm.at[pl.ds(toff, CHUNK)], dstb.at[0])
    pltpu.sync_copy(w_hbm.at[pl.ds(toff, CHUNK)], wbuf.at[0])
    pltpu.async_copy(xw_hbm.at[srcb.at[0]], rows2.at[1], gsem).wait()
    scale_chunk(1, 0)
    pltpu.sync_copy(rows2.at[1], aggr_sh.at[dstb.at[0]], add=True)

    plsc.subcore_barrier()
    pltpu.sync_copy(aggr_sh.at[pl.ds(s * N_PER, N_PER)],
                    p_hbm.at[c, pl.ds(s * N_PER, N_PER)])


_aggr = functools.partial(
    pl.kernel,
    out_type=jax.ShapeDtypeStruct((NUM_CORES, N_PAD, CH), _f32),
    mesh=_SC_MESH,
    compiler_params=pltpu.CompilerParams(needs_layout_passes=False),
    scratch_types=[
        pltpu.VMEM((4, CHUNK), _i32),
        pltpu.VMEM((4, CHUNK), _i32),
        pltpu.VMEM((4, CHUNK), _f32),
        pltpu.VMEM((2, CHUNK, CH), _f32),
        pltpu.VMEM_SHARED((N_PAD, CH), _f32),
        pltpu.SemaphoreType.DMA,
        pltpu.SemaphoreType.DMA,
        pltpu.SemaphoreType.DMA,
    ],
)(_aggr_body)


@jax.jit
def kernel(x, edge_index, edge_attr, W, We, att, bias):
    src = edge_index[0].astype(_i32).reshape(NW, NCHUNK, CHUNK)
    dst = edge_index[1].astype(_i32).reshape(NW, NCHUNK, CHUNK)
    attf = att.reshape(2 * CH + 4)
    a2 = jnp.pad(jnp.stack([attf[:CH], attf[CH:2 * CH]], axis=1),
                 ((0, 0), (0, 6)))
    wet = jnp.pad(We.T, ((0, 4), (0, 0)))
    ae = jnp.pad(attf[2 * CH:].reshape(4, 1), ((0, 4), (0, 0)))

    xw, s, smax = _k1(x, W, a2)
    eat, se, semax = _k2(edge_attr.T, wet, ae)
    ea = eat.T
    s_i = s[:, 0]
    s_j = s[:, 1]

    t = smax[0, 0] + smax[0, 1] + semax[0, 0]
    m = jnp.where(t >= 0.0, t, NEG_SLOPE * t)
    m_arr = jnp.full((16,), m, dtype=_f32)
    zn = jnp.zeros((N_NODES,), dtype=_f32)
    zr = jnp.zeros((N_PER, CH), dtype=_f32)

    w2d, dpart = _attn(s_i, s_j, se, src, dst, m_arr, zn)
    parts = _aggr(xw, src.reshape(-1), dst.reshape(-1), w2d.reshape(-1), zr)
    out = _k7(parts, dpart.T, bias.reshape(1, CH))
    return out, edge_index, ea


# speedup vs baseline: 1.9467x; 1.0343x over previous
import functools

import jax
import jax.numpy as jnp
from jax import lax
from jax.experimental import pallas as pl
from jax.experimental.pallas import tpu as pltpu
from jax.experimental.pallas import tpu_sc as plsc

N_NODES = 10000
N_EDGES = 320000
CH = 128
NEG_SLOPE = 0.2

NUM_CORES = 2
NUM_SUBCORES = 16
NW = NUM_CORES * NUM_SUBCORES
E_PER = N_EDGES // NW
CHUNK = 80
NCHUNK = E_PER // CHUNK
N_PAD = 10112
N_PER = N_PAD // NUM_SUBCORES

_f32 = jnp.float32
_i32 = jnp.int32


def _k1_body(x_ref, w_ref, a2_ref, xw_ref, s_ref, smax_ref):
    xw = jnp.dot(x_ref[...], w_ref[...], preferred_element_type=_f32)
    xw_ref[...] = xw
    s = jnp.dot(xw, a2_ref[...], preferred_element_type=_f32)
    s_ref[...] = s
    smax_ref[...] = jnp.max(s, axis=0, keepdims=True)


_k1 = pl.pallas_call(
    _k1_body,
    out_shape=(
        jax.ShapeDtypeStruct((N_NODES, CH), _f32),
        jax.ShapeDtypeStruct((N_NODES, 8), _f32),
        jax.ShapeDtypeStruct((1, 8), _f32),
    ),
)

def _k2_body(eat_ref, wet_ref, ae_ref, eat_out_ref, se_ref, semax_ref):
    eat = jnp.dot(wet_ref[...], eat_ref[...], preferred_element_type=_f32)
    eat_out_ref[...] = eat[:4]
    se = jnp.sum(eat * ae_ref[...], axis=0)
    se_ref[...] = se
    semax_ref[...] = jnp.full((1, 8), jnp.max(se), dtype=_f32)


_k2 = pl.pallas_call(
    _k2_body,
    out_shape=(
        jax.ShapeDtypeStruct((4, N_EDGES), _f32),
        jax.ShapeDtypeStruct((N_EDGES,), _f32),
        jax.ShapeDtypeStruct((1, 8), _f32),
    ),
)


def _k7_body(p_ref, dpt_ref, b_ref, o_ref):
    den = dpt_ref[:, 0:1] + dpt_ref[:, 1:2] + 1e-16
    o_ref[...] = (p_ref[0, :N_NODES] + p_ref[1, :N_NODES]) / den + b_ref[...]


_k7 = pl.pallas_call(
    _k7_body,
    out_shape=jax.ShapeDtypeStruct((N_NODES, CH), _f32),
)


_SC_MESH = plsc.VectorSubcoreMesh(core_axis_name="c", subcore_axis_name="s")


def _attn_body(si_hbm, sj_hbm, se_hbm, src_hbm, dst_hbm, m_hbm, zn_hbm,
               w_hbm, dpart_hbm,
               si_v, sj_v, se_v, src_v, dst_v, w_v, m_v, den_sh, dsem):
    c = lax.axis_index("c")
    s = lax.axis_index("s")
    wid = c * NUM_SUBCORES + s

    @pl.when(s == 0)
    def _():
        pltpu.sync_copy(zn_hbm, den_sh)

    pltpu.sync_copy(si_hbm, si_v)
    pltpu.sync_copy(sj_hbm, sj_v)
    pltpu.sync_copy(se_hbm.at[pl.ds(wid * E_PER, E_PER)], se_v)
    pltpu.sync_copy(src_hbm.at[wid], src_v)
    pltpu.sync_copy(dst_hbm.at[wid], dst_v)
    pltpu.sync_copy(m_hbm, m_v)
    gmax = m_v[...]
    plsc.subcore_barrier()

    def jbody(j, carry):
        for g in range(CHUNK // 16):
            sl = pl.ds(g * 16, 16)
            di = dst_v[j, sl]
            sri = src_v[j, sl]
            l = (plsc.load_gather(si_v, [di])
                 + plsc.load_gather(sj_v, [sri])
                 + se_v[pl.ds(j * CHUNK + g * 16, 16)])
            l = jnp.where(l >= 0.0, l, l * NEG_SLOPE)
            w_v[j, sl] = jnp.exp(l - gmax)
        pltpu.async_copy(w_v.at[j], den_sh.at[dst_v.at[j]], dsem, add=True)
        return carry

    lax.fori_loop(0, NCHUNK, jbody, 0)
    pltpu.sync_copy(w_v, w_hbm.at[wid])

    def dwait(j, carry):
        pltpu.make_async_copy(w_v.at[0], den_sh.at[dst_v.at[0]], dsem).wait()
        return carry

    lax.fori_loop(0, NCHUNK, dwait, 0)
    plsc.subcore_barrier()

    @pl.when(s == 0)
    def _():
        pltpu.sync_copy(den_sh, dpart_hbm.at[c])


_attn = functools.partial(
    pl.kernel,
    out_type=(
        jax.ShapeDtypeStruct((NW, NCHUNK, CHUNK), _f32),
        jax.ShapeDtypeStruct((NUM_CORES, N_NODES), _f32),
    ),
    mesh=_SC_MESH,
    compiler_params=pltpu.CompilerParams(needs_layout_passes=False),
    scratch_types=[
        pltpu.VMEM((N_NODES,), _f32),
        pltpu.VMEM((N_NODES,), _f32),
        pltpu.VMEM((E_PER,), _f32),
        pltpu.VMEM((NCHUNK, CHUNK), _i32),
        pltpu.VMEM((NCHUNK, CHUNK), _i32),
        pltpu.VMEM((NCHUNK, CHUNK), _f32),
        pltpu.VMEM((16,), _f32),
        pltpu.VMEM_SHARED((N_NODES,), _f32),
        pltpu.SemaphoreType.DMA,
    ],
)(_attn_body)


NSUP = (NCHUNK - 1) // 2


def _aggr_body(xw_hbm, src_hbm, dst_hbm, w_hbm, zr_hbm,
               p_hbm,
               srcb, dstb, wbuf, rows2, aggr_sh, gsem, ssem, psem):
    c = lax.axis_index("c")
    s = lax.axis_index("s")
    wid = c * NUM_SUBCORES + s
    ebase = wid * E_PER

    pltpu.sync_copy(zr_hbm, aggr_sh.at[pl.ds(s * N_PER, N_PER)])
    plsc.subcore_barrier()

    def scale_chunk(k, wslot):
        def gbody(g, gcarry):
            w16 = wbuf[wslot, pl.ds(g * 16, 16)]
            for kk in range(16):
                a = w16[kk]
                row = g * 16 + kk
                for f in range(CH // 16):
                    fl = pl.ds(f * 16, 16)
                    rows2[k, row, fl] = rows2[k, row, fl] * a
            return gcarry

        lax.fori_loop(0, CHUNK // 16, gbody, 0)

    def fetch_idx(row, slot, issue):
        off = ebase + row * CHUNK
        if issue:
            pltpu.async_copy(src_hbm.at[pl.ds(off, CHUNK)], srcb.at[slot],
                             psem)
            pltpu.async_copy(dst_hbm.at[pl.ds(off, CHUNK)], dstb.at[slot],
                             psem)
            pltpu.async_copy(w_hbm.at[pl.ds(off, CHUNK)], wbuf.at[slot],
                             psem)
        else:
            pltpu.make_async_copy(src_hbm.at[pl.ds(0, CHUNK)], srcb.at[slot],
                                  psem).wait()
            pltpu.make_async_copy(dst_hbm.at[pl.ds(0, CHUNK)], dstb.at[slot],
                                  psem).wait()
            pltpu.make_async_copy(w_hbm.at[pl.ds(0, CHUNK)], wbuf.at[slot],
                                  psem).wait()

    fetch_idx(0, 0, True)
    fetch_idx(1, 1, True)

    def jbody(J, carry):
        pb = lax.rem(J, 2)
        pn = 1 - pb
        sA = 2 * pb
        sB = 2 * pb + 1
        nA = 2 * pn
        nB = 2 * pn + 1
        nxt = jnp.minimum(2 * J + 2, NCHUNK - 2)
        fetch_idx(0, sA, False)
        fetch_idx(0, sB, False)

        @pl.when(J > 0)
        def _():
            pltpu.make_async_copy(rows2.at[0], aggr_sh.at[dstb.at[nA]],
                                  ssem).wait()
            pltpu.make_async_copy(rows2.at[1], aggr_sh.at[dstb.at[nB]],
                                  ssem).wait()

        pltpu.async_copy(xw_hbm.at[srcb.at[sA]], rows2.at[0], gsem)
        pltpu.async_copy(xw_hbm.at[srcb.at[sB]], rows2.at[1], gsem)
        fetch_idx(nxt, nA, True)
        fetch_idx(nxt + 1, nB, True)

        pltpu.make_async_copy(xw_hbm.at[srcb.at[sA]], rows2.at[0],
                              gsem).wait()
        scale_chunk(0, sA)
        pltpu.async_copy(rows2.at[0], aggr_sh.at[dstb.at[sA]], ssem,
                         add=True)
        pltpu.make_async_copy(xw_hbm.at[srcb.at[sB]], rows2.at[1],
                              gsem).wait()
        scale_chunk(1, sB)
        pltpu.async_copy(rows2.at[1], aggr_sh.at[dstb.at[sB]], ssem,
                         add=True)
        return carry

    lax.fori_loop(0, NSUP, jbody, 0)
    lastA = 2 * lax.rem(NSUP - 1, 2)
    pltpu.make_async_copy(rows2.at[0], aggr_sh.at[dstb.at[lastA]],
                          ssem).wait()
    pltpu.make_async_copy(rows2.at[1], aggr_sh.at[dstb.at[lastA + 1]],
                          ssem).wait()
    fetch_idx(0, 0, False)
    fetch_idx(0, 1, False)

    toff = ebase + (NCHUNK - 1) * CHUNK
    pltpu.sync_copy(src_hbm.at[pl.ds(toff, CHUNK)], srcb.at[0])
    pltpu.sync_copy(dst_hbm.at[pl.ds(toff, CHUNK)], dstb.at[0])
    pltpu.sync_copy(w_hbm.at[pl.ds(toff, CHUNK)], wbuf.at[0])
    pltpu.async_copy(xw_hbm.at[srcb.at[0]], rows2.at[1], gsem).wait()
    scale_chunk(1, 0)
    pltpu.sync_copy(rows2.at[1], aggr_sh.at[dstb.at[0]], add=True)

    plsc.subcore_barrier()
    pltpu.sync_copy(aggr_sh.at[pl.ds(s * N_PER, N_PER)],
                    p_hbm.at[c, pl.ds(s * N_PER, N_PER)])


_aggr = functools.partial(
    pl.kernel,
    out_type=jax.ShapeDtypeStruct((NUM_CORES, N_PAD, CH), _f32),
    mesh=_SC_MESH,
    compiler_params=pltpu.CompilerParams(needs_layout_passes=False),
    scratch_types=[
        pltpu.VMEM((4, CHUNK), _i32),
        pltpu.VMEM((4, CHUNK), _i32),
        pltpu.VMEM((4, CHUNK), _f32),
        pltpu.VMEM((2, CHUNK, CH), _f32),
        pltpu.VMEM_SHARED((N_PAD, CH), _f32),
        pltpu.SemaphoreType.DMA,
        pltpu.SemaphoreType.DMA,
        pltpu.SemaphoreType.DMA,
    ],
)(_aggr_body)


@jax.jit
def kernel(x, edge_index, edge_attr, W, We, att, bias):
    src = edge_index[0].astype(_i32).reshape(NW, NCHUNK, CHUNK)
    dst = edge_index[1].astype(_i32).reshape(NW, NCHUNK, CHUNK)
    attf = att.reshape(2 * CH + 4)
    a2 = jnp.pad(jnp.stack([attf[:CH], attf[CH:2 * CH]], axis=1),
                 ((0, 0), (0, 6)))
    wet = jnp.pad(We.T, ((0, 4), (0, 0)))
    ae = jnp.pad(attf[2 * CH:].reshape(4, 1), ((0, 4), (0, 0)))

    xw, s, smax = _k1(x, W, a2)
    eat, se, semax = _k2(edge_attr.T, wet, ae)
    ea = eat.T
    s_i = s[:, 0]
    s_j = s[:, 1]

    t = smax[0, 0] + smax[0, 1] + semax[0, 0]
    m = jnp.where(t >= 0.0, t, NEG_SLOPE * t)
    m_arr = jnp.full((16,), m, dtype=_f32)
    zn = jnp.zeros((N_NODES,), dtype=_f32)
    zr = jnp.zeros((N_PER, CH), dtype=_f32)

    w2d, dpart = _attn(s_i, s_j, se, src, dst, m_arr, zn)
    parts = _aggr(xw, src.reshape(-1), dst.reshape(-1), w2d.reshape(-1), zr)
    out = _k7(parts, dpart.T, bias.reshape(1, CH))
    return out, edge_index, ea


# speedup vs baseline: 2.1014x; 1.0795x over previous
import functools

import jax
import jax.numpy as jnp
from jax import lax
from jax.experimental import pallas as pl
from jax.experimental.pallas import tpu as pltpu
from jax.experimental.pallas import tpu_sc as plsc

N_NODES = 10000
N_EDGES = 320000
CH = 128
NEG_SLOPE = 0.2

NUM_CORES = 2
NUM_SUBCORES = 16
NW = NUM_CORES * NUM_SUBCORES
E_PER = N_EDGES // NW
CHUNK = 80
NCHUNK = E_PER // CHUNK
N_PAD = 10112
N_PER = N_PAD // NUM_SUBCORES

_f32 = jnp.float32
_i32 = jnp.int32


def _k1_body(x_ref, w_ref, a2_ref, xw_ref, s_ref, smax_ref):
    xw = jnp.dot(x_ref[...], w_ref[...], preferred_element_type=_f32)
    xw_ref[...] = xw
    s = jnp.dot(xw, a2_ref[...], preferred_element_type=_f32)
    s_ref[...] = s
    smax_ref[...] = jnp.max(s, axis=0, keepdims=True)


_k1 = pl.pallas_call(
    _k1_body,
    out_shape=(
        jax.ShapeDtypeStruct((N_NODES, CH), _f32),
        jax.ShapeDtypeStruct((N_NODES, 8), _f32),
        jax.ShapeDtypeStruct((1, 8), _f32),
    ),
)

def _k2_body(eat_ref, wet_ref, ae_ref, eat_out_ref, se_ref, semax_ref):
    eat = jnp.dot(wet_ref[...], eat_ref[...], preferred_element_type=_f32)
    eat_out_ref[...] = eat[:4]
    se = jnp.sum(eat * ae_ref[...], axis=0)
    se_ref[...] = se
    semax_ref[...] = jnp.full((1, 8), jnp.max(se), dtype=_f32)


_k2 = pl.pallas_call(
    _k2_body,
    out_shape=(
        jax.ShapeDtypeStruct((4, N_EDGES), _f32),
        jax.ShapeDtypeStruct((N_EDGES,), _f32),
        jax.ShapeDtypeStruct((1, 8), _f32),
    ),
)


def _k7_body(p_ref, dpt_ref, b_ref, o_ref):
    den = dpt_ref[:, 0:1] + dpt_ref[:, 1:2] + 1e-16
    o_ref[...] = (p_ref[0, :N_NODES] + p_ref[1, :N_NODES]) / den + b_ref[...]


_k7 = pl.pallas_call(
    _k7_body,
    out_shape=jax.ShapeDtypeStruct((N_NODES, CH), _f32),
)


_SC_MESH = plsc.VectorSubcoreMesh(core_axis_name="c", subcore_axis_name="s")


def _attn_body(si_hbm, sj_hbm, se_hbm, src_hbm, dst_hbm, m_hbm, zn_hbm,
               w_hbm, dpart_hbm,
               si_v, sj_v, se_v, src_v, dst_v, w_v, m_v, den_sh, dsem):
    c = lax.axis_index("c")
    s = lax.axis_index("s")
    wid = c * NUM_SUBCORES + s

    @pl.when(s == 0)
    def _():
        pltpu.sync_copy(zn_hbm, den_sh)

    pltpu.sync_copy(si_hbm, si_v)
    pltpu.sync_copy(sj_hbm, sj_v)
    pltpu.sync_copy(se_hbm.at[pl.ds(wid * E_PER, E_PER)], se_v)
    pltpu.sync_copy(src_hbm.at[wid], src_v)
    pltpu.sync_copy(dst_hbm.at[wid], dst_v)
    pltpu.sync_copy(m_hbm, m_v)
    gmax = m_v[...]
    plsc.subcore_barrier()

    def jbody(j, carry):
        for g in range(CHUNK // 16):
            sl = pl.ds(g * 16, 16)
            di = dst_v[j, sl]
            sri = src_v[j, sl]
            l = (plsc.load_gather(si_v, [di])
                 + plsc.load_gather(sj_v, [sri])
                 + se_v[pl.ds(j * CHUNK + g * 16, 16)])
            l = jnp.where(l >= 0.0, l, l * NEG_SLOPE)
            w_v[j, sl] = jnp.exp(l - gmax)
        pltpu.async_copy(w_v.at[j], den_sh.at[dst_v.at[j]], dsem, add=True)
        return carry

    lax.fori_loop(0, NCHUNK, jbody, 0)
    pltpu.sync_copy(w_v, w_hbm.at[wid])

    def dwait(j, carry):
        pltpu.make_async_copy(w_v.at[0], den_sh.at[dst_v.at[0]], dsem).wait()
        return carry

    lax.fori_loop(0, NCHUNK, dwait, 0)
    plsc.subcore_barrier()

    @pl.when(s == 0)
    def _():
        pltpu.sync_copy(den_sh, dpart_hbm.at[c])


_attn = functools.partial(
    pl.kernel,
    out_type=(
        jax.ShapeDtypeStruct((NW, NCHUNK, CHUNK), _f32),
        jax.ShapeDtypeStruct((NUM_CORES, N_NODES), _f32),
    ),
    mesh=_SC_MESH,
    compiler_params=pltpu.CompilerParams(needs_layout_passes=False),
    scratch_types=[
        pltpu.VMEM((N_NODES,), _f32),
        pltpu.VMEM((N_NODES,), _f32),
        pltpu.VMEM((E_PER,), _f32),
        pltpu.VMEM((NCHUNK, CHUNK), _i32),
        pltpu.VMEM((NCHUNK, CHUNK), _i32),
        pltpu.VMEM((NCHUNK, CHUNK), _f32),
        pltpu.VMEM((16,), _f32),
        pltpu.VMEM_SHARED((N_NODES,), _f32),
        pltpu.SemaphoreType.DMA,
    ],
)(_attn_body)


NSUP = (NCHUNK - 1) // 2


def _aggr_body(xw_hbm, src_hbm, dst_hbm, w_hbm, zr_hbm,
               p_hbm,
               srcb, dstb, wbuf, rows2, aggr_sh, gsem, ssem, psem):
    c = lax.axis_index("c")
    s = lax.axis_index("s")
    wid = c * NUM_SUBCORES + s
    ebase = wid * E_PER

    pltpu.sync_copy(zr_hbm, aggr_sh.at[pl.ds(s * N_PER, N_PER)])
    plsc.subcore_barrier()

    def scale_chunk(k, wslot):
        def gbody(g, gcarry):
            w16 = wbuf[wslot, pl.ds(g * 16, 16)]
            for kk in range(16):
                a = w16[kk]
                row = g * 16 + kk
                for f in range(CH // 16):
                    fl = pl.ds(f * 16, 16)
                    rows2[k, row, fl] = rows2[k, row, fl] * a
            return gcarry

        lax.fori_loop(0, CHUNK // 16, gbody, 0)

    def fetch_idx(row, slot, issue):
        off = ebase + row * CHUNK
        if issue:
            pltpu.async_copy(src_hbm.at[pl.ds(off, CHUNK)], srcb.at[slot],
                             psem)
            pltpu.async_copy(dst_hbm.at[pl.ds(off, CHUNK)], dstb.at[slot],
                             psem)
            pltpu.async_copy(w_hbm.at[pl.ds(off, CHUNK)], wbuf.at[slot],
                             psem)
        else:
            pltpu.make_async_copy(src_hbm.at[pl.ds(0, CHUNK)], srcb.at[slot],
                                  psem).wait()
            pltpu.make_async_copy(dst_hbm.at[pl.ds(0, CHUNK)], dstb.at[slot],
                                  psem).wait()
            pltpu.make_async_copy(w_hbm.at[pl.ds(0, CHUNK)], wbuf.at[slot],
                                  psem).wait()

    fetch_idx(0, 0, True)
    fetch_idx(1, 1, True)

    def jbody(J, carry):
        pb = lax.rem(J, 2)
        pn = 1 - pb
        sA = 2 * pb
        sB = 2 * pb + 1
        nA = 2 * pn
        nB = 2 * pn + 1
        nxt = jnp.minimum(2 * J + 2, NCHUNK - 2)
        fetch_idx(0, sA, False)
        fetch_idx(0, sB, False)

        @pl.when(J > 0)
        def _():
            pltpu.make_async_copy(rows2.at[0], aggr_sh.at[dstb.at[nA]],
                                  ssem).wait()

        pltpu.async_copy(xw_hbm.at[srcb.at[sA]], rows2.at[0], gsem)
        fetch_idx(nxt, nA, True)
        fetch_idx(nxt + 1, nB, True)

        @pl.when(J > 0)
        def _():
            pltpu.make_async_copy(rows2.at[1], aggr_sh.at[dstb.at[nB]],
                                  ssem).wait()

        pltpu.async_copy(xw_hbm.at[srcb.at[sB]], rows2.at[1], gsem)

        pltpu.make_async_copy(xw_hbm.at[srcb.at[sA]], rows2.at[0],
                              gsem).wait()
        scale_chunk(0, sA)
        pltpu.async_copy(rows2.at[0], aggr_sh.at[dstb.at[sA]], ssem,
                         add=True)
        pltpu.make_async_copy(xw_hbm.at[srcb.at[sB]], rows2.at[1],
                              gsem).wait()
        scale_chunk(1, sB)
        pltpu.async_copy(rows2.at[1], aggr_sh.at[dstb.at[sB]], ssem,
                         add=True)
        return carry

    lax.fori_loop(0, NSUP, jbody, 0)
    lastA = 2 * lax.rem(NSUP - 1, 2)
    pltpu.make_async_copy(rows2.at[0], aggr_sh.at[dstb.at[lastA]],
                          ssem).wait()
    pltpu.make_async_copy(rows2.at[1], aggr_sh.at[dstb.at[lastA + 1]],
                          ssem).wait()
    fetch_idx(0, 0, False)
    fetch_idx(0, 1, False)

    toff = ebase + (NCHUNK - 1) * CHUNK
    pltpu.sync_copy(src_hbm.at[pl.ds(toff, CHUNK)], srcb.at[0])
    pltpu.sync_copy(dst_hbm.at[pl.ds(toff, CHUNK)], dstb.at[0])
    pltpu.sync_copy(w_hbm.at[pl.ds(toff, CHUNK)], wbuf.at[0])
    pltpu.async_copy(xw_hbm.at[srcb.at[0]], rows2.at[1], gsem).wait()
    scale_chunk(1, 0)
    pltpu.sync_copy(rows2.at[1], aggr_sh.at[dstb.at[0]], add=True)

    plsc.subcore_barrier()
    pltpu.sync_copy(aggr_sh.at[pl.ds(s * N_PER, N_PER)],
                    p_hbm.at[c, pl.ds(s * N_PER, N_PER)])


_aggr = functools.partial(
    pl.kernel,
    out_type=jax.ShapeDtypeStruct((NUM_CORES, N_PAD, CH), _f32),
    mesh=_SC_MESH,
    compiler_params=pltpu.CompilerParams(needs_layout_passes=False),
    scratch_types=[
        pltpu.VMEM((4, CHUNK), _i32),
        pltpu.VMEM((4, CHUNK), _i32),
        pltpu.VMEM((4, CHUNK), _f32),
        pltpu.VMEM((2, CHUNK, CH), _f32),
        pltpu.VMEM_SHARED((N_PAD, CH), _f32),
        pltpu.SemaphoreType.DMA,
        pltpu.SemaphoreType.DMA,
        pltpu.SemaphoreType.DMA,
    ],
)(_aggr_body)


@jax.jit
def kernel(x, edge_index, edge_attr, W, We, att, bias):
    src = edge_index[0].astype(_i32).reshape(NW, NCHUNK, CHUNK)
    dst = edge_index[1].astype(_i32).reshape(NW, NCHUNK, CHUNK)
    attf = att.reshape(2 * CH + 4)
    a2 = jnp.pad(jnp.stack([attf[:CH], attf[CH:2 * CH]], axis=1),
                 ((0, 0), (0, 6)))
    wet = jnp.pad(We.T, ((0, 4), (0, 0)))
    ae = jnp.pad(attf[2 * CH:].reshape(4, 1), ((0, 4), (0, 0)))

    xw, s, smax = _k1(x, W, a2)
    eat, se, semax = _k2(edge_attr.T, wet, ae)
    ea = eat.T
    s_i = s[:, 0]
    s_j = s[:, 1]

    t = smax[0, 0] + smax[0, 1] + semax[0, 0]
    m = jnp.where(t >= 0.0, t, NEG_SLOPE * t)
    m_arr = jnp.full((16,), m, dtype=_f32)
    zn = jnp.zeros((N_NODES,), dtype=_f32)
    zr = jnp.zeros((N_PER, CH), dtype=_f32)

    w2d, dpart = _attn(s_i, s_j, se, src, dst, m_arr, zn)
    parts = _aggr(xw, src.reshape(-1), dst.reshape(-1), w2d.reshape(-1), zr)
    out = _k7(parts, dpart.T, bias.reshape(1, CH))
    return out, edge_index, ea


# speedup vs baseline: 2.1328x; 1.0149x over previous
import functools

import jax
import jax.numpy as jnp
from jax import lax
from jax.experimental import pallas as pl
from jax.experimental.pallas import tpu as pltpu
from jax.experimental.pallas import tpu_sc as plsc

N_NODES = 10000
N_EDGES = 320000
CH = 128
NEG_SLOPE = 0.2

NUM_CORES = 2
NUM_SUBCORES = 16
NW = NUM_CORES * NUM_SUBCORES
E_PER = N_EDGES // NW
CHUNK = 80
NCHUNK = E_PER // CHUNK
N_PAD = 10112
N_PER = N_PAD // NUM_SUBCORES

_f32 = jnp.float32
_i32 = jnp.int32


def _k1_body(x_ref, w_ref, ai_ref, aj_ref, xw_ref, si_ref, sj_ref,
             smax_ref):
    xw = jnp.dot(x_ref[...], w_ref[...], preferred_element_type=_f32)
    xw_ref[...] = xw
    si = jnp.sum(xw * ai_ref[...], axis=-1)
    sj = jnp.sum(xw * aj_ref[...], axis=-1)
    si_ref[...] = si
    sj_ref[...] = sj
    smax_ref[...] = jnp.full((1, 8), jnp.max(si) + jnp.max(sj), dtype=_f32)


_k1 = pl.pallas_call(
    _k1_body,
    out_shape=(
        jax.ShapeDtypeStruct((N_NODES, CH), _f32),
        jax.ShapeDtypeStruct((N_NODES,), _f32),
        jax.ShapeDtypeStruct((N_NODES,), _f32),
        jax.ShapeDtypeStruct((1, 8), _f32),
    ),
)

def _k2_body(eat_ref, wet_ref, ae_ref, eat_out_ref, se_ref, semax_ref):
    eat = jnp.dot(wet_ref[...], eat_ref[...], preferred_element_type=_f32)
    eat_out_ref[...] = eat[:4]
    se = jnp.sum(eat * ae_ref[...], axis=0)
    se_ref[...] = se
    semax_ref[...] = jnp.full((1, 8), jnp.max(se), dtype=_f32)


_k2 = pl.pallas_call(
    _k2_body,
    out_shape=(
        jax.ShapeDtypeStruct((4, N_EDGES), _f32),
        jax.ShapeDtypeStruct((N_EDGES,), _f32),
        jax.ShapeDtypeStruct((1, 8), _f32),
    ),
)


def _k7_body(p_ref, dpt_ref, b_ref, o_ref):
    den = dpt_ref[:, 0:1] + dpt_ref[:, 1:2] + 1e-16
    o_ref[...] = (p_ref[0, :N_NODES] + p_ref[1, :N_NODES]) / den + b_ref[...]


_k7 = pl.pallas_call(
    _k7_body,
    out_shape=jax.ShapeDtypeStruct((N_NODES, CH), _f32),
)


_SC_MESH = plsc.VectorSubcoreMesh(core_axis_name="c", subcore_axis_name="s")


def _attn_body(si_hbm, sj_hbm, se_hbm, src_hbm, dst_hbm, m_hbm, zn_hbm,
               w_hbm, dpart_hbm,
               si_v, sj_v, se_v, src_v, dst_v, w_v, m_v, den_sh, dsem):
    c = lax.axis_index("c")
    s = lax.axis_index("s")
    wid = c * NUM_SUBCORES + s
    base = wid * E_PER

    @pl.when(s == 0)
    def _():
        pltpu.sync_copy(zn_hbm, den_sh)

    pltpu.sync_copy(si_hbm, si_v)
    pltpu.sync_copy(sj_hbm, sj_v)
    pltpu.sync_copy(se_hbm.at[pl.ds(wid * E_PER, E_PER)], se_v)
    pltpu.sync_copy(src_hbm.at[wid], src_v)
    pltpu.sync_copy(dst_hbm.at[wid], dst_v)
    pltpu.sync_copy(m_hbm, m_v)
    gmax = m_v[...]
    plsc.subcore_barrier()

    def jbody(j, carry):
        for g in range(CHUNK // 16):
            sl = pl.ds(g * 16, 16)
            di = dst_v[j, sl]
            sri = src_v[j, sl]
            l = (plsc.load_gather(si_v, [di])
                 + plsc.load_gather(sj_v, [sri])
                 + se_v[pl.ds(j * CHUNK + g * 16, 16)])
            l = jnp.where(l >= 0.0, l, l * NEG_SLOPE)
            w_v[pl.ds(j * CHUNK + g * 16, 16)] = jnp.exp(l - gmax)
        pltpu.async_copy(w_v.at[pl.ds(j * CHUNK, CHUNK)],
                         den_sh.at[dst_v.at[j]], dsem, add=True)
        return carry

    lax.fori_loop(0, NCHUNK, jbody, 0)
    pltpu.sync_copy(w_v, w_hbm.at[pl.ds(base, E_PER)])

    def dwait(j, carry):
        pltpu.make_async_copy(w_v.at[pl.ds(0, CHUNK)],
                              den_sh.at[dst_v.at[0]], dsem).wait()
        return carry

    lax.fori_loop(0, NCHUNK, dwait, 0)
    plsc.subcore_barrier()

    @pl.when(s == 0)
    def _():
        pltpu.sync_copy(den_sh, dpart_hbm.at[c])


_attn = functools.partial(
    pl.kernel,
    out_type=(
        jax.ShapeDtypeStruct((N_EDGES,), _f32),
        jax.ShapeDtypeStruct((NUM_CORES, N_NODES), _f32),
    ),
    mesh=_SC_MESH,
    compiler_params=pltpu.CompilerParams(needs_layout_passes=False),
    scratch_types=[
        pltpu.VMEM((N_NODES,), _f32),
        pltpu.VMEM((N_NODES,), _f32),
        pltpu.VMEM((E_PER,), _f32),
        pltpu.VMEM((NCHUNK, CHUNK), _i32),
        pltpu.VMEM((NCHUNK, CHUNK), _i32),
        pltpu.VMEM((E_PER,), _f32),
        pltpu.VMEM((16,), _f32),
        pltpu.VMEM_SHARED((N_NODES,), _f32),
        pltpu.SemaphoreType.DMA,
    ],
)(_attn_body)


NSUP = (NCHUNK - 1) // 2


def _aggr_body(xw_hbm, src_hbm, dst_hbm, w_hbm, zr_hbm,
               p_hbm,
               srcb, dstb, wbuf, rows2, aggr_sh, gsem, ssem, psem):
    c = lax.axis_index("c")
    s = lax.axis_index("s")
    wid = c * NUM_SUBCORES + s
    ebase = wid * E_PER

    pltpu.sync_copy(zr_hbm, aggr_sh.at[pl.ds(s * N_PER, N_PER)])
    plsc.subcore_barrier()

    def scale_chunk(k, wslot):
        def gbody(g, gcarry):
            w16 = wbuf[wslot, pl.ds(g * 16, 16)]
            for kk in range(16):
                a = w16[kk]
                row = g * 16 + kk
                for f in range(CH // 16):
                    fl = pl.ds(f * 16, 16)
                    rows2[k, row, fl] = rows2[k, row, fl] * a
            return gcarry

        lax.fori_loop(0, CHUNK // 16, gbody, 0)

    def fetch_idx(row, slot, issue):
        off = ebase + row * CHUNK
        if issue:
            pltpu.async_copy(src_hbm.at[pl.ds(off, CHUNK)], srcb.at[slot],
                             psem)
            pltpu.async_copy(dst_hbm.at[pl.ds(off, CHUNK)], dstb.at[slot],
                             psem)
            pltpu.async_copy(w_hbm.at[pl.ds(off, CHUNK)], wbuf.at[slot],
                             psem)
        else:
            pltpu.make_async_copy(src_hbm.at[pl.ds(0, CHUNK)], srcb.at[slot],
                                  psem).wait()
            pltpu.make_async_copy(dst_hbm.at[pl.ds(0, CHUNK)], dstb.at[slot],
                                  psem).wait()
            pltpu.make_async_copy(w_hbm.at[pl.ds(0, CHUNK)], wbuf.at[slot],
                                  psem).wait()

    fetch_idx(0, 0, True)
    fetch_idx(1, 1, True)

    def jbody(J, carry):
        pb = lax.rem(J, 2)
        pn = 1 - pb
        sA = 2 * pb
        sB = 2 * pb + 1
        nA = 2 * pn
        nB = 2 * pn + 1
        nxt = jnp.minimum(2 * J + 2, NCHUNK - 2)
        fetch_idx(0, sA, False)
        fetch_idx(0, sB, False)

        @pl.when(J > 0)
        def _():
            pltpu.make_async_copy(rows2.at[0], aggr_sh.at[dstb.at[nA]],
                                  ssem).wait()

        pltpu.async_copy(xw_hbm.at[srcb.at[sA]], rows2.at[0], gsem)
        fetch_idx(nxt, nA, True)
        fetch_idx(nxt + 1, nB, True)

        @pl.when(J > 0)
        def _():
            pltpu.make_async_copy(rows2.at[1], aggr_sh.at[dstb.at[nB]],
                                  ssem).wait()

        pltpu.async_copy(xw_hbm.at[srcb.at[sB]], rows2.at[1], gsem)

        pltpu.make_async_copy(xw_hbm.at[srcb.at[sA]], rows2.at[0],
                              gsem).wait()
        scale_chunk(0, sA)
        pltpu.async_copy(rows2.at[0], aggr_sh.at[dstb.at[sA]], ssem,
                         add=True)
        pltpu.make_async_copy(xw_hbm.at[srcb.at[sB]], rows2.at[1],
                              gsem).wait()
        scale_chunk(1, sB)
        pltpu.async_copy(rows2.at[1], aggr_sh.at[dstb.at[sB]], ssem,
                         add=True)
        return carry

    lax.fori_loop(0, NSUP, jbody, 0)
    lastA = 2 * lax.rem(NSUP - 1, 2)
    pltpu.make_async_copy(rows2.at[0], aggr_sh.at[dstb.at[lastA]],
                          ssem).wait()
    pltpu.make_async_copy(rows2.at[1], aggr_sh.at[dstb.at[lastA + 1]],
                          ssem).wait()
    fetch_idx(0, 0, False)
    fetch_idx(0, 1, False)

    toff = ebase + (NCHUNK - 1) * CHUNK
    pltpu.sync_copy(src_hbm.at[pl.ds(toff, CHUNK)], srcb.at[0])
    pltpu.sync_copy(dst_hbm.at[pl.ds(toff, CHUNK)], dstb.at[0])
    pltpu.sync_copy(w_hbm.at[pl.ds(toff, CHUNK)], wbuf.at[0])
    pltpu.async_copy(xw_hbm.at[srcb.at[0]], rows2.at[1], gsem).wait()
    scale_chunk(1, 0)
    pltpu.sync_copy(rows2.at[1], aggr_sh.at[dstb.at[0]], add=True)

    plsc.subcore_barrier()
    pltpu.sync_copy(aggr_sh.at[pl.ds(s * N_PER, N_PER)],
                    p_hbm.at[c, pl.ds(s * N_PER, N_PER)])


_aggr = functools.partial(
    pl.kernel,
    out_type=jax.ShapeDtypeStruct((NUM_CORES, N_PAD, CH), _f32),
    mesh=_SC_MESH,
    compiler_params=pltpu.CompilerParams(needs_layout_passes=False),
    scratch_types=[
        pltpu.VMEM((4, CHUNK), _i32),
        pltpu.VMEM((4, CHUNK), _i32),
        pltpu.VMEM((4, CHUNK), _f32),
        pltpu.VMEM((2, CHUNK, CH), _f32),
        pltpu.VMEM_SHARED((N_PAD, CH), _f32),
        pltpu.SemaphoreType.DMA,
        pltpu.SemaphoreType.DMA,
        pltpu.SemaphoreType.DMA,
    ],
)(_aggr_body)


@jax.jit
def kernel(x, edge_index, edge_attr, W, We, att, bias):
    src = edge_index[0].astype(_i32).reshape(NW, NCHUNK, CHUNK)
    dst = edge_index[1].astype(_i32).reshape(NW, NCHUNK, CHUNK)
    attf = att.reshape(2 * CH + 4)
    wet = jnp.pad(We.T, ((0, 4), (0, 0)))
    ae = jnp.pad(attf[2 * CH:].reshape(4, 1), ((0, 4), (0, 0)))

    xw, s_i, s_j, smax = _k1(x, W, attf[:CH].reshape(1, CH),
                             attf[CH:2 * CH].reshape(1, CH))
    eat, se, semax = _k2(edge_attr.T, wet, ae)
    ea = eat.T

    t = smax[0, 0] + semax[0, 0]
    m = jnp.where(t >= 0.0, t, NEG_SLOPE * t)
    m_arr = jnp.full((16,), m, dtype=_f32)
    zn = jnp.zeros((N_NODES,), dtype=_f32)
    zr = jnp.zeros((N_PER, CH), dtype=_f32)

    w2d, dpart = _attn(s_i, s_j, se, src, dst, m_arr, zn)
    parts = _aggr(xw, src.reshape(-1), dst.reshape(-1), w2d, zr)
    out = _k7(parts, dpart.T, bias.reshape(1, CH))
    return out, edge_index, ea
